# baseline (speedup 1.0000x reference)
import jax
import jax.numpy as jnp
from jax.experimental import pallas as pl
from jax.experimental.pallas import tpu as pltpu

_BM_F32 = 384
_BM_BF16 = 1024


def _xw_body(x_ref, w_ref, y_ref):
    y_ref[...] = jnp.dot(
        x_ref[...].astype(jnp.bfloat16), w_ref[...],
        preferred_element_type=jnp.float32,
    ).astype(jnp.bfloat16)


def _feature_matmul(x, w_bf16, bm):
    n, d_in = x.shape
    d_out = w_bf16.shape[1]
    return pl.pallas_call(
        _xw_body,
        grid=(n // bm,),
        in_specs=[
            pl.BlockSpec((bm, d_in), lambda i: (i, 0)),
            pl.BlockSpec((d_in, d_out), lambda i: (0, 0)),
        ],
        out_specs=pl.BlockSpec((bm, d_out), lambda i: (i, 0)),
        out_shape=jax.ShapeDtypeStruct((n, d_out), jnp.bfloat16),
    )(x, w_bf16)


def _layer1_body(adj_ref, x_ref, w1_ref, b_ref, wn_ref, abf_ref, yn_ref,
                 y1_ref):
    i = pl.program_id(0)

    @pl.when(i == 0)
    def _compute_y1():
        y1_ref[...] = jnp.dot(
            x_ref[...].astype(jnp.bfloat16), w1_ref[...],
            preferred_element_type=jnp.float32,
        ).astype(jnp.bfloat16)

    a = adj_ref[...].astype(jnp.bfloat16)
    abf_ref[...] = a
    h = jnp.dot(a, y1_ref[...], preferred_element_type=jnp.float32) + b_ref[...]
    r = jnp.maximum(h, 0.0).astype(jnp.bfloat16)
    yn_ref[...] = jnp.dot(
        r, wn_ref[...], preferred_element_type=jnp.float32
    ).astype(jnp.bfloat16)


def _layer1(adj, x, w1_bf16, b1_row, w2_bf16):
    n = adj.shape[0]
    d_in = x.shape[1]
    d = w1_bf16.shape[1]
    dn = w2_bf16.shape[1]
    bm = _BM_F32
    return pl.pallas_call(
        _layer1_body,
        grid=(pl.cdiv(n, bm),),
        in_specs=[
            pl.BlockSpec((bm, n), lambda i: (i, 0)),
            pl.BlockSpec((n, d_in), lambda i: (0, 0)),
            pl.BlockSpec((d_in, d), lambda i: (0, 0)),
            pl.BlockSpec((1, d), lambda i: (0, 0)),
            pl.BlockSpec((d, dn), lambda i: (0, 0)),
        ],
        out_specs=[
            pl.BlockSpec((bm, n), lambda i: (i, 0)),
            pl.BlockSpec((bm, dn), lambda i: (i, 0)),
        ],
        out_shape=[
            jax.ShapeDtypeStruct((n, n), jnp.bfloat16),
            jax.ShapeDtypeStruct((n, dn), jnp.bfloat16),
        ],
        scratch_shapes=[pltpu.VMEM((n, d), jnp.bfloat16)],
        compiler_params=pltpu.CompilerParams(
            dimension_semantics=("arbitrary",),
        ),
    )(adj, x, w1_bf16, b1_row, w2_bf16)


def _make_layer_body(cast_adj, emit_raw, emit_next):
    def body(*refs):
        it = iter(refs)
        adj_ref = next(it)
        y_ref = next(it)
        b_ref = next(it)
        wn_ref = next(it) if emit_next else None
        abf_ref = next(it) if cast_adj else None
        raw_ref = next(it) if emit_raw else None
        yn_ref = next(it) if emit_next else None

        a = adj_ref[...]
        if cast_adj:
            a = a.astype(jnp.bfloat16)
            abf_ref[...] = a

        h = jnp.dot(a, y_ref[...], preferred_element_type=jnp.float32)
        h = h + b_ref[...]
        if emit_raw:
            raw_ref[...] = h
        if emit_next:
            r = jnp.maximum(h, 0.0).astype(jnp.bfloat16)
            yn_ref[...] = jnp.dot(
                r, wn_ref[...], preferred_element_type=jnp.float32
            ).astype(jnp.bfloat16)

    return body


def _gcn_layer(adj, y, b_row, w_next=None, *, cast_adj=False, emit_raw=False):
    n = adj.shape[0]
    d = y.shape[1]
    bm = _BM_F32 if cast_adj else _BM_BF16
    emit_next = w_next is not None

    in_specs = [
        pl.BlockSpec((bm, n), lambda i: (i, 0)),
        pl.BlockSpec((n, d), lambda i: (0, 0)),
        pl.BlockSpec((1, d), lambda i: (0, 0)),
    ]
    operands = [adj, y, b_row]
    if emit_next:
        dn = w_next.shape[1]
        in_specs.append(pl.BlockSpec((d, dn), lambda i: (0, 0)))
        operands.append(w_next)

    out_shape = []
    out_specs = []
    if cast_adj:
        out_shape.append(jax.ShapeDtypeStruct((n, n), jnp.bfloat16))
        out_specs.append(pl.BlockSpec((bm, n), lambda i: (i, 0)))
    if emit_raw:
        out_shape.append(jax.ShapeDtypeStruct((n, d), jnp.float32))
        out_specs.append(pl.BlockSpec((bm, d), lambda i: (i, 0)))
    if emit_next:
        out_shape.append(jax.ShapeDtypeStruct((n, dn), jnp.bfloat16))
        out_specs.append(pl.BlockSpec((bm, dn), lambda i: (i, 0)))

    outs = pl.pallas_call(
        _make_layer_body(cast_adj, emit_raw, emit_next),
        grid=(pl.cdiv(n, bm),),
        in_specs=in_specs,
        out_specs=out_specs,
        out_shape=out_shape,
        compiler_params=pltpu.CompilerParams(
            dimension_semantics=("arbitrary",),
        ),
    )(*operands)
    return outs


def kernel(x, adj, W1, b1, W2, b2, W3, b3, W4, b4):
    W1b = W1.astype(jnp.bfloat16)
    W2b = W2.astype(jnp.bfloat16)
    W3b = W3.astype(jnp.bfloat16)
    W4b = W4.astype(jnp.bfloat16)
    b1r = b1.reshape(1, -1)
    b2r = b2.reshape(1, -1)
    b3r = b3.reshape(1, -1)
    b4r = b4.reshape(1, -1)

    adj_bf, y2 = _layer1(adj, x, W1b, b1r, W2b)
    x_out, y3 = _gcn_layer(adj_bf, y2, b2r, W3b, emit_raw=True)
    (y4,) = _gcn_layer(adj_bf, y3, b3r, W4b)
    (x_rec,) = _gcn_layer(adj_bf, y4, b4r, emit_raw=True)
    return (x_out, x_rec)


# speedup vs baseline: 1.0341x; 1.0341x over previous
import jax
import jax.numpy as jnp
from jax.experimental import pallas as pl
from jax.experimental.pallas import tpu as pltpu

_BM_F32 = 384
_BM_BF16 = 1024


def _xw_body(x_ref, w_ref, y_ref):
    y_ref[...] = jnp.dot(
        x_ref[...].astype(jnp.bfloat16), w_ref[...],
        preferred_element_type=jnp.float32,
    ).astype(jnp.bfloat16)


def _feature_matmul(x, w_bf16, bm):
    n, d_in = x.shape
    d_out = w_bf16.shape[1]
    return pl.pallas_call(
        _xw_body,
        grid=(n // bm,),
        in_specs=[
            pl.BlockSpec((bm, d_in), lambda i: (i, 0)),
            pl.BlockSpec((d_in, d_out), lambda i: (0, 0)),
        ],
        out_specs=pl.BlockSpec((bm, d_out), lambda i: (i, 0)),
        out_shape=jax.ShapeDtypeStruct((n, d_out), jnp.bfloat16),
    )(x, w_bf16)


def _layer1_body(adj_ref, x_ref, w1_ref, b_ref, wn_ref, abf_ref, yn_ref,
                 y1_ref):
    i = pl.program_id(0)

    @pl.when(i == 0)
    def _compute_y1():
        y1_ref[...] = jnp.dot(
            x_ref[...].astype(jnp.bfloat16), w1_ref[...],
            preferred_element_type=jnp.float32,
        ).astype(jnp.bfloat16)

    a = adj_ref[...].astype(jnp.bfloat16)
    abf_ref[...] = a
    h = jnp.dot(a, y1_ref[...], preferred_element_type=jnp.float32) + b_ref[...]
    r = jnp.maximum(h, 0.0).astype(jnp.bfloat16)
    yn_ref[...] = jnp.dot(
        r, wn_ref[...], preferred_element_type=jnp.float32
    ).astype(jnp.bfloat16)


def _layer1(adj, x, w1_bf16, b1_row, w2_bf16):
    n = adj.shape[0]
    d_in = x.shape[1]
    d = w1_bf16.shape[1]
    dn = w2_bf16.shape[1]
    bm = _BM_F32
    return pl.pallas_call(
        _layer1_body,
        grid=(pl.cdiv(n, bm),),
        in_specs=[
            pl.BlockSpec((bm, n), lambda i: (i, 0)),
            pl.BlockSpec((n, d_in), lambda i: (0, 0)),
            pl.BlockSpec((d_in, d), lambda i: (0, 0)),
            pl.BlockSpec((1, d), lambda i: (0, 0)),
            pl.BlockSpec((d, dn), lambda i: (0, 0)),
        ],
        out_specs=[
            pl.BlockSpec((bm, n), lambda i: (i, 0)),
            pl.BlockSpec((bm, dn), lambda i: (i, 0)),
        ],
        out_shape=[
            jax.ShapeDtypeStruct((n, n), jnp.bfloat16),
            jax.ShapeDtypeStruct((n, dn), jnp.bfloat16),
        ],
        scratch_shapes=[pltpu.VMEM((n, d), jnp.bfloat16)],
        compiler_params=pltpu.CompilerParams(
            dimension_semantics=("arbitrary",),
        ),
    )(adj, x, w1_bf16, b1_row, w2_bf16)


def _make_layer_body(cast_adj, emit_raw, emit_next):
    def body(*refs):
        it = iter(refs)
        adj_ref = next(it)
        y_ref = next(it)
        b_ref = next(it)
        wn_ref = next(it) if emit_next else None
        abf_ref = next(it) if cast_adj else None
        raw_ref = next(it) if emit_raw else None
        yn_ref = next(it) if emit_next else None

        a = adj_ref[...]
        if cast_adj:
            a = a.astype(jnp.bfloat16)
            abf_ref[...] = a

        h = jnp.dot(a, y_ref[...], preferred_element_type=jnp.float32)
        h = h + b_ref[...]
        if emit_raw:
            raw_ref[...] = h
        if emit_next:
            r = jnp.maximum(h, 0.0).astype(jnp.bfloat16)
            yn_ref[...] = jnp.dot(
                r, wn_ref[...], preferred_element_type=jnp.float32
            ).astype(jnp.bfloat16)

    return body


def _gcn_layer(adj, y, b_row, w_next=None, *, cast_adj=False, emit_raw=False):
    n = adj.shape[0]
    d = y.shape[1]
    bm = _BM_F32 if cast_adj else _BM_BF16
    emit_next = w_next is not None

    in_specs = [
        pl.BlockSpec((bm, n), lambda i: (i, 0)),
        pl.BlockSpec((n, d), lambda i: (0, 0)),
        pl.BlockSpec((1, d), lambda i: (0, 0)),
    ]
    operands = [adj, y, b_row]
    if emit_next:
        dn = w_next.shape[1]
        in_specs.append(pl.BlockSpec((d, dn), lambda i: (0, 0)))
        operands.append(w_next)

    out_shape = []
    out_specs = []
    if cast_adj:
        out_shape.append(jax.ShapeDtypeStruct((n, n), jnp.bfloat16))
        out_specs.append(pl.BlockSpec((bm, n), lambda i: (i, 0)))
    if emit_raw:
        out_shape.append(jax.ShapeDtypeStruct((n, d), jnp.float32))
        out_specs.append(pl.BlockSpec((bm, d), lambda i: (i, 0)))
    if emit_next:
        out_shape.append(jax.ShapeDtypeStruct((n, dn), jnp.bfloat16))
        out_specs.append(pl.BlockSpec((bm, dn), lambda i: (i, 0)))

    outs = pl.pallas_call(
        _make_layer_body(cast_adj, emit_raw, emit_next),
        grid=(pl.cdiv(n, bm),),
        in_specs=in_specs,
        out_specs=out_specs,
        out_shape=out_shape,
        compiler_params=pltpu.CompilerParams(
            dimension_semantics=("arbitrary",),
        ),
    )(*operands)
    return outs


def _make_decoder_body(n, nj, bm):
    def body(adj_ref, y2_ref, b2_ref, b3_ref, b4_ref, w3_ref, w4_ref,
             xout_ref, xrec_ref, y3_ref, y4_ref):
        pid = pl.program_id(0)
        phase = pid // nj
        j = pid % nj
        row0 = j * bm

        def phase_l2():
            h = jnp.dot(adj_ref[...], y2_ref[...],
                        preferred_element_type=jnp.float32) + b2_ref[...]
            xout_ref[...] = h
            r = jnp.maximum(h, 0.0).astype(jnp.bfloat16)
            y3_ref[pl.ds(row0, bm), :] = jnp.dot(
                r, w3_ref[...], preferred_element_type=jnp.float32
            ).astype(jnp.bfloat16)

        def phase_l3():
            h = jnp.dot(adj_ref[...], y3_ref[pl.ds(0, n), :],
                        preferred_element_type=jnp.float32) + b3_ref[...]
            r = jnp.maximum(h, 0.0).astype(jnp.bfloat16)
            y4_ref[pl.ds(row0, bm), :] = jnp.dot(
                r, w4_ref[...], preferred_element_type=jnp.float32
            ).astype(jnp.bfloat16)

        def phase_l4():
            xrec_ref[...] = jnp.dot(
                adj_ref[...], y4_ref[pl.ds(0, n), :],
                preferred_element_type=jnp.float32) + b4_ref[...]

        jax.lax.switch(phase, (phase_l2, phase_l3, phase_l4))

    return body


def _decoder_fused(adj_bf, y2, b2_row, b3_row, b4_row, w3_bf16, w4_bf16):
    n = adj_bf.shape[0]
    d2 = y2.shape[1]
    d3 = w3_bf16.shape[1]
    d4 = w4_bf16.shape[1]
    bm = _BM_BF16
    nj = pl.cdiv(n, bm)

    def adj_map(pid):
        return (pid % nj, 0)

    def xout_map(pid):
        phase = pid // nj
        j = pid % nj
        return (jnp.where(phase == 0, j, nj - 1), 0)

    def xrec_map(pid):
        phase = pid // nj
        j = pid % nj
        return (jnp.where(phase == 2, j, 0), 0)

    return pl.pallas_call(
        _make_decoder_body(n, nj, bm),
        grid=(3 * nj,),
        in_specs=[
            pl.BlockSpec((bm, n), adj_map),
            pl.BlockSpec((n, d2), lambda pid: (0, 0)),
            pl.BlockSpec((1, d2), lambda pid: (0, 0)),
            pl.BlockSpec((1, d3), lambda pid: (0, 0)),
            pl.BlockSpec((1, d4), lambda pid: (0, 0)),
            pl.BlockSpec((d2, d3), lambda pid: (0, 0)),
            pl.BlockSpec((d3, d4), lambda pid: (0, 0)),
        ],
        out_specs=[
            pl.BlockSpec((bm, d2), xout_map),
            pl.BlockSpec((bm, d4), xrec_map),
        ],
        out_shape=[
            jax.ShapeDtypeStruct((n, d2), jnp.float32),
            jax.ShapeDtypeStruct((n, d4), jnp.float32),
        ],
        scratch_shapes=[
            pltpu.VMEM((nj * bm, d3), jnp.bfloat16),
            pltpu.VMEM((nj * bm, d4), jnp.bfloat16),
        ],
        compiler_params=pltpu.CompilerParams(
            dimension_semantics=("arbitrary",),
        ),
    )(adj_bf, y2, b2_row, b3_row, b4_row, w3_bf16, w4_bf16)


def kernel(x, adj, W1, b1, W2, b2, W3, b3, W4, b4):
    W1b = W1.astype(jnp.bfloat16)
    W2b = W2.astype(jnp.bfloat16)
    W3b = W3.astype(jnp.bfloat16)
    W4b = W4.astype(jnp.bfloat16)
    b1r = b1.reshape(1, -1)
    b2r = b2.reshape(1, -1)
    b3r = b3.reshape(1, -1)
    b4r = b4.reshape(1, -1)

    adj_bf, y2 = _layer1(adj, x, W1b, b1r, W2b)
    x_out, x_rec = _decoder_fused(adj_bf, y2, b2r, b3r, b4r, W3b, W4b)
    return (x_out, x_rec)


# speedup vs baseline: 1.0440x; 1.0095x over previous
import jax
import jax.numpy as jnp
from jax.experimental import pallas as pl
from jax.experimental.pallas import tpu as pltpu

_BM_F32 = 384
_BM_BF16 = 1152


def _xw_body(x_ref, w_ref, y_ref):
    y_ref[...] = jnp.dot(
        x_ref[...].astype(jnp.bfloat16), w_ref[...],
        preferred_element_type=jnp.float32,
    ).astype(jnp.bfloat16)


def _feature_matmul(x, w_bf16, bm):
    n, d_in = x.shape
    d_out = w_bf16.shape[1]
    return pl.pallas_call(
        _xw_body,
        grid=(n // bm,),
        in_specs=[
            pl.BlockSpec((bm, d_in), lambda i: (i, 0)),
            pl.BlockSpec((d_in, d_out), lambda i: (0, 0)),
        ],
        out_specs=pl.BlockSpec((bm, d_out), lambda i: (i, 0)),
        out_shape=jax.ShapeDtypeStruct((n, d_out), jnp.bfloat16),
    )(x, w_bf16)


def _layer1_body(adj_ref, x_ref, w1_ref, b_ref, wn_ref, abf_ref, yn_ref,
                 y1_ref):
    i = pl.program_id(0)

    @pl.when(i == 0)
    def _compute_y1():
        y1_ref[...] = jnp.dot(
            x_ref[...].astype(jnp.bfloat16), w1_ref[...],
            preferred_element_type=jnp.float32,
        ).astype(jnp.bfloat16)

    a = adj_ref[...].astype(jnp.bfloat16)
    abf_ref[...] = a
    h = jnp.dot(a, y1_ref[...], preferred_element_type=jnp.float32) + b_ref[...]
    r = jnp.maximum(h, 0.0).astype(jnp.bfloat16)
    yn_ref[...] = jnp.dot(
        r, wn_ref[...], preferred_element_type=jnp.float32
    ).astype(jnp.bfloat16)


def _layer1(adj, x, w1_bf16, b1_row, w2_bf16):
    n = adj.shape[0]
    d_in = x.shape[1]
    d = w1_bf16.shape[1]
    dn = w2_bf16.shape[1]
    bm = _BM_F32
    return pl.pallas_call(
        _layer1_body,
        grid=(pl.cdiv(n, bm),),
        in_specs=[
            pl.BlockSpec((bm, n), lambda i: (i, 0)),
            pl.BlockSpec((n, d_in), lambda i: (0, 0)),
            pl.BlockSpec((d_in, d), lambda i: (0, 0)),
            pl.BlockSpec((1, d), lambda i: (0, 0)),
            pl.BlockSpec((d, dn), lambda i: (0, 0)),
        ],
        out_specs=[
            pl.BlockSpec((bm, n), lambda i: (i, 0)),
            pl.BlockSpec((bm, dn), lambda i: (i, 0)),
        ],
        out_shape=[
            jax.ShapeDtypeStruct((n, n), jnp.bfloat16),
            jax.ShapeDtypeStruct((n, dn), jnp.bfloat16),
        ],
        scratch_shapes=[pltpu.VMEM((n, d), jnp.bfloat16)],
        compiler_params=pltpu.CompilerParams(
            dimension_semantics=("arbitrary",),
        ),
    )(adj, x, w1_bf16, b1_row, w2_bf16)


def _make_layer_body(cast_adj, emit_raw, emit_next):
    def body(*refs):
        it = iter(refs)
        adj_ref = next(it)
        y_ref = next(it)
        b_ref = next(it)
        wn_ref = next(it) if emit_next else None
        abf_ref = next(it) if cast_adj else None
        raw_ref = next(it) if emit_raw else None
        yn_ref = next(it) if emit_next else None

        a = adj_ref[...]
        if cast_adj:
            a = a.astype(jnp.bfloat16)
            abf_ref[...] = a

        h = jnp.dot(a, y_ref[...], preferred_element_type=jnp.float32)
        h = h + b_ref[...]
        if emit_raw:
            raw_ref[...] = h
        if emit_next:
            r = jnp.maximum(h, 0.0).astype(jnp.bfloat16)
            yn_ref[...] = jnp.dot(
                r, wn_ref[...], preferred_element_type=jnp.float32
            ).astype(jnp.bfloat16)

    return body


def _gcn_layer(adj, y, b_row, w_next=None, *, cast_adj=False, emit_raw=False):
    n = adj.shape[0]
    d = y.shape[1]
    bm = _BM_F32 if cast_adj else _BM_BF16
    emit_next = w_next is not None

    in_specs = [
        pl.BlockSpec((bm, n), lambda i: (i, 0)),
        pl.BlockSpec((n, d), lambda i: (0, 0)),
        pl.BlockSpec((1, d), lambda i: (0, 0)),
    ]
    operands = [adj, y, b_row]
    if emit_next:
        dn = w_next.shape[1]
        in_specs.append(pl.BlockSpec((d, dn), lambda i: (0, 0)))
        operands.append(w_next)

    out_shape = []
    out_specs = []
    if cast_adj:
        out_shape.append(jax.ShapeDtypeStruct((n, n), jnp.bfloat16))
        out_specs.append(pl.BlockSpec((bm, n), lambda i: (i, 0)))
    if emit_raw:
        out_shape.append(jax.ShapeDtypeStruct((n, d), jnp.float32))
        out_specs.append(pl.BlockSpec((bm, d), lambda i: (i, 0)))
    if emit_next:
        out_shape.append(jax.ShapeDtypeStruct((n, dn), jnp.bfloat16))
        out_specs.append(pl.BlockSpec((bm, dn), lambda i: (i, 0)))

    outs = pl.pallas_call(
        _make_layer_body(cast_adj, emit_raw, emit_next),
        grid=(pl.cdiv(n, bm),),
        in_specs=in_specs,
        out_specs=out_specs,
        out_shape=out_shape,
        compiler_params=pltpu.CompilerParams(
            dimension_semantics=("arbitrary",),
        ),
    )(*operands)
    return outs


def _make_decoder_body(n, nj, bm):
    def body(adj_ref, y2_ref, b2_ref, b3_ref, b4_ref, w3_ref, w4_ref,
             xout_ref, xrec_ref, y3_ref, y4_ref):
        pid = pl.program_id(0)
        phase = pid // nj
        j = pid % nj
        row0 = j * bm

        def phase_l2():
            h = jnp.dot(adj_ref[...], y2_ref[...],
                        preferred_element_type=jnp.float32) + b2_ref[...]
            xout_ref[...] = h
            r = jnp.maximum(h, 0.0).astype(jnp.bfloat16)
            y3_ref[pl.ds(row0, bm), :] = jnp.dot(
                r, w3_ref[...], preferred_element_type=jnp.float32
            ).astype(jnp.bfloat16)

        def phase_l3():
            h = jnp.dot(adj_ref[...], y3_ref[pl.ds(0, n), :],
                        preferred_element_type=jnp.float32) + b3_ref[...]
            r = jnp.maximum(h, 0.0).astype(jnp.bfloat16)
            y4_ref[pl.ds(row0, bm), :] = jnp.dot(
                r, w4_ref[...], preferred_element_type=jnp.float32
            ).astype(jnp.bfloat16)

        def phase_l4():
            xrec_ref[...] = jnp.dot(
                adj_ref[...], y4_ref[pl.ds(0, n), :],
                preferred_element_type=jnp.float32) + b4_ref[...]

        jax.lax.switch(phase, (phase_l2, phase_l3, phase_l4))

    return body


def _decoder_fused(adj_bf, y2, b2_row, b3_row, b4_row, w3_bf16, w4_bf16):
    n = adj_bf.shape[0]
    d2 = y2.shape[1]
    d3 = w3_bf16.shape[1]
    d4 = w4_bf16.shape[1]
    bm = _BM_BF16
    nj = pl.cdiv(n, bm)

    def adj_map(pid):
        return (pid % nj, 0)

    def xout_map(pid):
        phase = pid // nj
        j = pid % nj
        return (jnp.where(phase == 0, j, nj - 1), 0)

    def xrec_map(pid):
        phase = pid // nj
        j = pid % nj
        return (jnp.where(phase == 2, j, 0), 0)

    return pl.pallas_call(
        _make_decoder_body(n, nj, bm),
        grid=(3 * nj,),
        in_specs=[
            pl.BlockSpec((bm, n), adj_map),
            pl.BlockSpec((n, d2), lambda pid: (0, 0)),
            pl.BlockSpec((1, d2), lambda pid: (0, 0)),
            pl.BlockSpec((1, d3), lambda pid: (0, 0)),
            pl.BlockSpec((1, d4), lambda pid: (0, 0)),
            pl.BlockSpec((d2, d3), lambda pid: (0, 0)),
            pl.BlockSpec((d3, d4), lambda pid: (0, 0)),
        ],
        out_specs=[
            pl.BlockSpec((bm, d2), xout_map),
            pl.BlockSpec((bm, d4), xrec_map),
        ],
        out_shape=[
            jax.ShapeDtypeStruct((n, d2), jnp.float32),
            jax.ShapeDtypeStruct((n, d4), jnp.float32),
        ],
        scratch_shapes=[
            pltpu.VMEM((nj * bm, d3), jnp.bfloat16),
            pltpu.VMEM((nj * bm, d4), jnp.bfloat16),
        ],
        compiler_params=pltpu.CompilerParams(
            dimension_semantics=("arbitrary",),
        ),
    )(adj_bf, y2, b2_row, b3_row, b4_row, w3_bf16, w4_bf16)


def kernel(x, adj, W1, b1, W2, b2, W3, b3, W4, b4):
    W1b = W1.astype(jnp.bfloat16)
    W2b = W2.astype(jnp.bfloat16)
    W3b = W3.astype(jnp.bfloat16)
    W4b = W4.astype(jnp.bfloat16)
    b1r = b1.reshape(1, -1)
    b2r = b2.reshape(1, -1)
    b3r = b3.reshape(1, -1)
    b4r = b4.reshape(1, -1)

    adj_bf, y2 = _layer1(adj, x, W1b, b1r, W2b)
    x_out, x_rec = _decoder_fused(adj_bf, y2, b2r, b3r, b4r, W3b, W4b)
    return (x_out, x_rec)
